# speedup vs baseline: 1.0044x; 1.0044x over previous
import jax
import jax.numpy as jnp
from jax.experimental import pallas as pl


def _final_body(pooled_ref, wlin_ref, blin_ref, out_ref):
    out_ref[...] = pooled_ref[...] @ wlin_ref[...] + blin_ref[...]


def kernel(x, edge_index, batch, W1_l, b1, W1_r, W2_l, b2, W2_r, Wlin, blin):
    N = x.shape[0]
    G = 64
    src = edge_index[0]
    dst = edge_index[1]

    def sage(xx, W_l, b, W_r):
        msgs = jnp.take(xx, src, axis=0)
        agg = jax.ops.segment_sum(msgs, dst, num_segments=N)
        deg = jax.ops.segment_sum(jnp.ones((dst.shape[0],), jnp.float32), dst,
                                  num_segments=N)
        agg = agg / jnp.clip(deg, 1.0)[:, None]
        return agg @ W_l + b + xx @ W_r

    h = jax.nn.relu(sage(x, W1_l, b1, W1_r))
    h = sage(h, W2_l, b2, W2_r)
    pooled_sum = jax.ops.segment_sum(h, batch, num_segments=G)
    counts = jax.ops.segment_sum(jnp.ones((N,), jnp.float32), batch,
                                 num_segments=G)
    pooled = pooled_sum / jnp.clip(counts, 1.0)[:, None]
    out = pl.pallas_call(
        _final_body,
        out_shape=jax.ShapeDtypeStruct((G, Wlin.shape[1]), jnp.float32),
    )(pooled, Wlin, blin)
    return out


# speedup vs baseline: 7.6231x; 7.5896x over previous
import jax
import jax.numpy as jnp
from jax import lax
from jax.experimental import pallas as pl
from jax.experimental.pallas import tpu as pltpu
from jax.experimental.pallas import tpu_sc as plsc

N = 50000
NPAD = 50048
E = 800000
F_IN = 26
FP = 32
H = 64
G = 64
NC = 2
NS = 16
NW = NC * NS
CB = 128
NCHUNKS = E // CB
ROWS_PER_TILE = NPAD // NS
ZROWS = 136
R = 3128
LANES = 16


def _l1_body(xpad, src, dst, out, acc, zbuf, srcb, dstb, msgs, gsem):
    c = lax.axis_index("c")
    s = lax.axis_index("s")
    w = c * NS + s
    zero16 = jnp.zeros((LANES,), jnp.float32)
    for r in range(ZROWS):
        for j in range(FP // LANES):
            zbuf[r, pl.ds(j * LANES, LANES)] = zero16

    def zloop(i, carry):
        pltpu.sync_copy(
            zbuf, acc.at[pl.ds(s * ROWS_PER_TILE + i * ZROWS, ZROWS), :])
        return carry

    lax.fori_loop(0, ROWS_PER_TILE // ZROWS, zloop, 0)
    plsc.subcore_barrier()

    nmine = NCHUNKS // NW + jnp.where(w < NCHUNKS % NW, 1, 0)

    def body(i, carry):
        off = (w + i * NW) * CB
        pltpu.sync_copy(src.at[pl.ds(off, CB)], srcb)
        pltpu.sync_copy(dst.at[pl.ds(off, CB)], dstb)
        pltpu.async_copy(xpad.at[srcb], msgs, gsem).wait()
        pltpu.sync_copy(msgs, acc.at[dstb], add=True)
        return carry

    lax.fori_loop(0, nmine, body, 0)
    plsc.subcore_barrier()
    pltpu.sync_copy(
        acc.at[pl.ds(s * ROWS_PER_TILE, ROWS_PER_TILE), :],
        out.at[c, pl.ds(s * ROWS_PER_TILE, ROWS_PER_TILE), :])


def _l2_body(h, src, dst, wvec, gvec, out, acc, wtab, gtab, zbuf,
             srcb, dstb, gbuf, msgs, gsem):
    c = lax.axis_index("c")
    s = lax.axis_index("s")
    w = c * NS + s
    zero16 = jnp.zeros((LANES,), jnp.float32)
    for r in range(8):
        for j in range(H // LANES):
            zbuf[r, pl.ds(j * LANES, LANES)] = zero16

    @pl.when(s < G // 8)
    def _():
        pltpu.sync_copy(zbuf, acc.at[pl.ds(s * 8, 8), :])

    pltpu.sync_copy(wvec, wtab)
    pltpu.sync_copy(gvec, gtab)
    plsc.subcore_barrier()

    nmine = NCHUNKS // NW + jnp.where(w < NCHUNKS % NW, 1, 0)

    def body(i, carry):
        off = (w + i * NW) * CB
        pltpu.sync_copy(src.at[pl.ds(off, CB)], srcb)
        pltpu.sync_copy(dst.at[pl.ds(off, CB)], dstb)
        pltpu.async_copy(h.at[srcb], msgs, gsem).wait()
        for k in range(CB // LANES):
            dv = dstb[pl.ds(k * LANES, LANES)]
            wv = plsc.load_gather(wtab, [dv])
            gv = plsc.load_gather(gtab, [dv])
            gbuf[pl.ds(k * LANES, LANES)] = gv
            for e in range(LANES):
                ws = jnp.broadcast_to(wv[e], (LANES,))
                row = k * LANES + e
                for j in range(H // LANES):
                    sl = pl.ds(j * LANES, LANES)
                    msgs[row, sl] = msgs[row, sl] * ws
        pltpu.sync_copy(msgs, acc.at[gbuf], add=True)
        return carry

    lax.fori_loop(0, nmine, body, 0)
    plsc.subcore_barrier()

    @pl.when(s == 0)
    def _():
        pltpu.sync_copy(acc, out.at[c])


def _tc1_body(p_ref, x_ref, batch_ref, wext_ref, h_ref, stats_ref):
    i = pl.program_id(0)
    agg = p_ref[0] + p_ref[1]
    deg = agg[:, 26:27]
    mean = agg / jnp.clip(deg, 1.0)
    inp = jnp.concatenate([mean, x_ref[...]], axis=1)
    hh = jnp.maximum(
        jnp.dot(inp, wext_ref[...], preferred_element_type=jnp.float32), 0.0)
    h_ref[...] = hh
    b = batch_ref[0, 0, :].reshape(R, 1)
    oh = (b == lax.broadcasted_iota(jnp.int32, (R, G), 1)).astype(jnp.float32)
    rhs = jnp.concatenate(
        [hh, jnp.ones((R, 1), jnp.float32),
         jnp.zeros((R, 128 - H - 1), jnp.float32)], axis=1)
    contrib = lax.dot_general(
        oh, rhs, (((0,), (0,)), ((), ())),
        preferred_element_type=jnp.float32)

    @pl.when(i == 0)
    def _():
        stats_ref[...] = jnp.zeros_like(stats_ref)

    stats_ref[...] += contrib


def _tc2_body(pool_ref, stats_ref, w2l_ref, b2_ref, w2r_ref, wlin_ref,
              blin_ref, out_ref):
    agg2 = pool_ref[0] + pool_ref[1]
    segh = stats_ref[:, :H]
    counts = stats_ref[:, H:H + 1]
    ps = (jnp.dot(agg2, w2l_ref[...], preferred_element_type=jnp.float32)
          + jnp.dot(segh, w2r_ref[...], preferred_element_type=jnp.float32)
          + counts * b2_ref[...])
    pooled = ps / jnp.clip(counts, 1.0)
    out_ref[...] = (
        jnp.dot(pooled, wlin_ref[...], preferred_element_type=jnp.float32)
        + blin_ref[...])


def kernel(x, edge_index, batch, W1_l, b1, W1_r, W2_l, b2, W2_r, Wlin, blin):
    src = edge_index[0].astype(jnp.int32)
    dst = edge_index[1].astype(jnp.int32)
    batch = batch.astype(jnp.int32)

    xpad = jnp.zeros((NPAD, FP), jnp.float32)
    xpad = xpad.at[:N, :F_IN].set(x).at[:N, F_IN].set(1.0)

    mesh = plsc.VectorSubcoreMesh(core_axis_name="c", subcore_axis_name="s")
    sc_params = pltpu.CompilerParams(use_tc_tiling_on_sc=False,
                                     needs_layout_passes=False)

    l1 = pl.kernel(
        _l1_body,
        out_type=jax.ShapeDtypeStruct((NC, NPAD, FP), jnp.float32),
        mesh=mesh,
        compiler_params=sc_params,
        scratch_types=[
            pltpu.VMEM_SHARED((NPAD, FP), jnp.float32),
            pltpu.VMEM((ZROWS, FP), jnp.float32),
            pltpu.VMEM((CB,), jnp.int32),
            pltpu.VMEM((CB,), jnp.int32),
            pltpu.VMEM((CB, FP), jnp.float32),
            pltpu.SemaphoreType.DMA,
        ],
    )
    partials = l1(xpad, src, dst)

    deg = partials[0, :N, F_IN] + partials[1, :N, F_IN]
    wvec = 1.0 / jnp.clip(deg, 1.0)

    wext = jnp.zeros((2 * FP, H), jnp.float32)
    wext = wext.at[:F_IN, :].set(W1_l)
    wext = wext.at[FP:FP + F_IN, :].set(W1_r)
    wext = wext.at[FP + F_IN, :].set(b1)

    batch_pad = jnp.full((NPAD,), G, jnp.int32).at[:N].set(batch)
    batch3d = batch_pad.reshape(NPAD // R, 1, R)

    h, stats = pl.pallas_call(
        _tc1_body,
        grid=(NPAD // R,),
        in_specs=[
            pl.BlockSpec((NC, R, FP), lambda i: (0, i, 0)),
            pl.BlockSpec((R, FP), lambda i: (i, 0)),
            pl.BlockSpec((1, 1, R), lambda i: (i, 0, 0)),
            pl.BlockSpec((2 * FP, H), lambda i: (0, 0)),
        ],
        out_specs=[
            pl.BlockSpec((R, H), lambda i: (i, 0)),
            pl.BlockSpec((G, 128), lambda i: (0, 0)),
        ],
        out_shape=[
            jax.ShapeDtypeStruct((NPAD, H), jnp.float32),
            jax.ShapeDtypeStruct((G, 128), jnp.float32),
        ],
    )(partials, xpad, batch3d, wext)

    l2 = pl.kernel(
        _l2_body,
        out_type=jax.ShapeDtypeStruct((NC, G, H), jnp.float32),
        mesh=mesh,
        compiler_params=sc_params,
        scratch_types=[
            pltpu.VMEM_SHARED((G, H), jnp.float32),
            pltpu.VMEM((N,), jnp.float32),
            pltpu.VMEM((N,), jnp.int32),
            pltpu.VMEM((8, H), jnp.float32),
            pltpu.VMEM((CB,), jnp.int32),
            pltpu.VMEM((CB,), jnp.int32),
            pltpu.VMEM((CB,), jnp.int32),
            pltpu.VMEM((CB, H), jnp.float32),
            pltpu.SemaphoreType.DMA,
        ],
    )
    pool = l2(h, src, dst, wvec, batch)

    out = pl.pallas_call(
        _tc2_body,
        out_shape=jax.ShapeDtypeStruct((G, F_IN), jnp.float32),
    )(pool, stats, W2_l, b2.reshape(1, H), W2_r, Wlin,
      blin.reshape(1, F_IN))
    return out


# speedup vs baseline: 14.6049x; 1.9159x over previous
import jax
import jax.numpy as jnp
from jax import lax
from jax.experimental import pallas as pl
from jax.experimental.pallas import tpu as pltpu
from jax.experimental.pallas import tpu_sc as plsc

N = 50000
NPAD = 50048
E = 800000
F_IN = 26
FP = 32
H = 64
G = 64
NC = 2
NS = 16
NW = NC * NS
CB = 128
NCHUNKS = E // CB
CPT = NCHUNKS // NW
NLEFT = NCHUNKS - CPT * NW
IB = 13
NBLK = CPT // IB
ROWS_PER_TILE = NPAD // NS
ZROWS = 68
R = 3128
LANES = 16


def _l1_body(xpad, src2d, dst2d, out, acc, zbuf, srcl, dstl, stail, dtail,
             msgs0, msgs1, sem0, sem1):
    c = lax.axis_index("c")
    s = lax.axis_index("s")
    w = c * NS + s

    def g_start(j, buf, sem):
        pltpu.async_copy(xpad.at[srcl.at[j]], buf, sem)

    def g_wait(j, buf, sem):
        pltpu.make_async_copy(xpad.at[srcl.at[j]], buf, sem).wait()

    zero16 = jnp.zeros((LANES,), jnp.float32)
    for r in range(ZROWS):
        for j in range(FP // LANES):
            zbuf[r, pl.ds(j * LANES, LANES)] = zero16

    def zloop(i, carry):
        pltpu.sync_copy(
            zbuf, acc.at[pl.ds(s * ROWS_PER_TILE + i * ZROWS, ZROWS), :])
        return carry

    lax.fori_loop(0, ROWS_PER_TILE // ZROWS, zloop, 0)
    plsc.subcore_barrier()

    def blk_body(blk, carry):
        cb0 = w * CPT + blk * IB
        pltpu.sync_copy(src2d.at[pl.ds(cb0, IB), :], srcl)
        pltpu.sync_copy(dst2d.at[pl.ds(cb0, IB), :], dstl)
        g_start(0, msgs0, sem0)

        def body(t, carry2):
            c0 = 2 * t
            c1 = 2 * t + 1
            g_start(c1, msgs1, sem1)
            g_wait(c0, msgs0, sem0)
            pltpu.sync_copy(msgs0, acc.at[dstl.at[c0]], add=True)
            g_start(c0 + 2, msgs0, sem0)
            g_wait(c1, msgs1, sem1)
            pltpu.sync_copy(msgs1, acc.at[dstl.at[c1]], add=True)
            return carry2

        lax.fori_loop(0, IB // 2, body, 0)
        g_wait(IB - 1, msgs0, sem0)
        pltpu.sync_copy(msgs0, acc.at[dstl.at[IB - 1]], add=True)
        return carry

    lax.fori_loop(0, NBLK, blk_body, 0)

    @pl.when(w < NLEFT)
    def _():
        pltpu.sync_copy(src2d.at[NW * CPT + w], stail)
        pltpu.sync_copy(dst2d.at[NW * CPT + w], dtail)
        pltpu.async_copy(xpad.at[stail], msgs1, sem1).wait()
        pltpu.sync_copy(msgs1, acc.at[dtail], add=True)

    plsc.subcore_barrier()
    pltpu.sync_copy(
        acc.at[pl.ds(s * ROWS_PER_TILE, ROWS_PER_TILE), :],
        out.at[c, pl.ds(s * ROWS_PER_TILE, ROWS_PER_TILE), :])


def _scale_rows(msgs, dstall, wgtab, gbuf, base):
    for k in range(CB // LANES):
        dv = dstall[pl.ds(base + k * LANES, LANES)]
        pk = plsc.load_gather(wgtab, [dv])
        gv = pk & jnp.int32(63)
        wv = plsc.bitcast(pk & jnp.int32(-64), jnp.float32)
        gbuf[pl.ds(k * LANES, LANES)] = gv
        for e in range(LANES):
            ws = jnp.broadcast_to(wv[e], (LANES,))
            row = k * LANES + e
            for j in range(H // LANES):
                sl = pl.ds(j * LANES, LANES)
                msgs[row, sl] = msgs[row, sl] * ws


def _l2_body(h, src2d, dst1d, wg, out, acc, wgtab, srcl, dstall, zbuf,
             gbuf, stail, msgs0, msgs1, sem0, sem1):
    c = lax.axis_index("c")
    s = lax.axis_index("s")
    w = c * NS + s

    pltpu.sync_copy(src2d.at[pl.ds(w * CPT, CPT), :], srcl)
    pltpu.sync_copy(dst1d.at[pl.ds(w * CPT * CB, CPT * CB)], dstall)
    pltpu.sync_copy(wg, wgtab)

    def g_start(j, buf, sem):
        pltpu.async_copy(h.at[srcl.at[j]], buf, sem)

    def g_wait(j, buf, sem):
        pltpu.make_async_copy(h.at[srcl.at[j]], buf, sem).wait()

    g_start(0, msgs0, sem0)

    zero16 = jnp.zeros((LANES,), jnp.float32)
    for r in range(8):
        for j in range(H // LANES):
            zbuf[r, pl.ds(j * LANES, LANES)] = zero16

    @pl.when(s < G // 8)
    def _():
        pltpu.sync_copy(zbuf, acc.at[pl.ds(s * 8, 8), :])

    plsc.subcore_barrier()

    def consume(j, buf, sem):
        g_wait(j, buf, sem)
        _scale_rows(buf, dstall, wgtab, gbuf, j * CB)
        pltpu.sync_copy(buf, acc.at[gbuf], add=True)

    def body(t, carry):
        c0 = 2 * t
        c1 = 2 * t + 1
        g_start(c1, msgs1, sem1)
        consume(c0, msgs0, sem0)

        @pl.when(c0 + 2 < CPT)
        def _():
            g_start(c0 + 2, msgs0, sem0)

        consume(c1, msgs1, sem1)
        return carry

    lax.fori_loop(0, CPT // 2, body, 0)
    consume(CPT - 1, msgs0, sem0)

    @pl.when(w < NLEFT)
    def _():
        pltpu.sync_copy(src2d.at[NW * CPT + w], stail)
        pltpu.sync_copy(
            dst1d.at[pl.ds((NW * CPT + w) * CB, CB)],
            dstall.at[pl.ds(0, CB)])
        pltpu.async_copy(h.at[stail], msgs1, sem1).wait()
        _scale_rows(msgs1, dstall, wgtab, gbuf, 0)
        pltpu.sync_copy(msgs1, acc.at[gbuf], add=True)

    plsc.subcore_barrier()

    @pl.when(s == 0)
    def _():
        pltpu.sync_copy(acc, out.at[c])


def _tc1_body(p_ref, x_ref, batch_ref, wext_ref, h_ref, stats_ref, wg_ref):
    i = pl.program_id(0)
    agg = p_ref[0] + p_ref[1]
    deg = agg[:, 26:27]
    mean = agg / jnp.clip(deg, 1.0)
    inp = jnp.concatenate([mean, x_ref[...]], axis=1)
    hh = jnp.maximum(
        jnp.dot(inp, wext_ref[...], preferred_element_type=jnp.float32), 0.0)
    h_ref[...] = hh
    b = batch_ref[0, 0, :].reshape(R, 1)
    oh = (b == lax.broadcasted_iota(jnp.int32, (R, G), 1)).astype(jnp.float32)
    rhs = jnp.concatenate(
        [hh, jnp.ones((R, 1), jnp.float32),
         jnp.zeros((R, 128 - H - 1), jnp.float32)], axis=1)
    contrib = lax.dot_general(
        oh, rhs, (((0,), (0,)), ((), ())),
        preferred_element_type=jnp.float32)

    @pl.when(i == 0)
    def _():
        stats_ref[...] = jnp.zeros_like(stats_ref)

    stats_ref[...] += contrib

    wbits = lax.bitcast_convert_type(1.0 / jnp.clip(deg, 1.0), jnp.int32)
    packed = (wbits & jnp.int32(-64)) | b
    wg_ref[0, 0, :] = packed[:, 0]


def _tc2_body(pool_ref, stats_ref, w2l_ref, b2_ref, w2r_ref, wlin_ref,
              blin_ref, out_ref):
    agg2 = pool_ref[0] + pool_ref[1]
    segh = stats_ref[:, :H]
    counts = stats_ref[:, H:H + 1]
    ps = (jnp.dot(agg2, w2l_ref[...], preferred_element_type=jnp.float32)
          + jnp.dot(segh, w2r_ref[...], preferred_element_type=jnp.float32)
          + counts * b2_ref[...])
    pooled = ps / jnp.clip(counts, 1.0)
    out_ref[...] = (
        jnp.dot(pooled, wlin_ref[...], preferred_element_type=jnp.float32)
        + blin_ref[...])


def kernel(x, edge_index, batch, W1_l, b1, W1_r, W2_l, b2, W2_r, Wlin, blin):
    src = edge_index[0].astype(jnp.int32)
    dst = edge_index[1].astype(jnp.int32)
    batch = batch.astype(jnp.int32)
    src2d = src.reshape(NCHUNKS, CB)
    dst2d = dst.reshape(NCHUNKS, CB)

    xpad = jnp.zeros((NPAD, FP), jnp.float32)
    xpad = xpad.at[:N, :F_IN].set(x).at[:N, F_IN].set(1.0)

    mesh = plsc.VectorSubcoreMesh(core_axis_name="c", subcore_axis_name="s")
    sc_params = pltpu.CompilerParams(use_tc_tiling_on_sc=False,
                                     needs_layout_passes=False)

    l1 = pl.kernel(
        _l1_body,
        out_type=jax.ShapeDtypeStruct((NC, NPAD, FP), jnp.float32),
        mesh=mesh,
        compiler_params=sc_params,
        scratch_types=[
            pltpu.VMEM_SHARED((NPAD, FP), jnp.float32),
            pltpu.VMEM((ZROWS, FP), jnp.float32),
            pltpu.VMEM((IB, CB), jnp.int32),
            pltpu.VMEM((IB, CB), jnp.int32),
            pltpu.VMEM((CB,), jnp.int32),
            pltpu.VMEM((CB,), jnp.int32),
            pltpu.VMEM((CB, FP), jnp.float32),
            pltpu.VMEM((CB, FP), jnp.float32),
            pltpu.SemaphoreType.DMA,
            pltpu.SemaphoreType.DMA,
        ],
    )
    partials = l1(xpad, src2d, dst2d)

    wext = jnp.zeros((2 * FP, H), jnp.float32)
    wext = wext.at[:F_IN, :].set(W1_l)
    wext = wext.at[FP:FP + F_IN, :].set(W1_r)
    wext = wext.at[FP + F_IN, :].set(b1)

    batch_pad = jnp.full((NPAD,), G, jnp.int32).at[:N].set(batch)
    batch3d = batch_pad.reshape(NPAD // R, 1, R)

    h, stats, wg3d = pl.pallas_call(
        _tc1_body,
        grid=(NPAD // R,),
        in_specs=[
            pl.BlockSpec((NC, R, FP), lambda i: (0, i, 0)),
            pl.BlockSpec((R, FP), lambda i: (i, 0)),
            pl.BlockSpec((1, 1, R), lambda i: (i, 0, 0)),
            pl.BlockSpec((2 * FP, H), lambda i: (0, 0)),
        ],
        out_specs=[
            pl.BlockSpec((R, H), lambda i: (i, 0)),
            pl.BlockSpec((G, 128), lambda i: (0, 0)),
            pl.BlockSpec((1, 1, R), lambda i: (i, 0, 0)),
        ],
        out_shape=[
            jax.ShapeDtypeStruct((NPAD, H), jnp.float32),
            jax.ShapeDtypeStruct((G, 128), jnp.float32),
            jax.ShapeDtypeStruct((NPAD // R, 1, R), jnp.int32),
        ],
    )(partials, xpad, batch3d, wext)
    wg = wg3d.reshape(NPAD)

    l2 = pl.kernel(
        _l2_body,
        out_type=jax.ShapeDtypeStruct((NC, G, H), jnp.float32),
        mesh=mesh,
        compiler_params=sc_params,
        scratch_types=[
            pltpu.VMEM_SHARED((G, H), jnp.float32),
            pltpu.VMEM((NPAD,), jnp.int32),
            pltpu.VMEM((CPT, CB), jnp.int32),
            pltpu.VMEM((CPT * CB,), jnp.int32),
            pltpu.VMEM((8, H), jnp.float32),
            pltpu.VMEM((CB,), jnp.int32),
            pltpu.VMEM((CB,), jnp.int32),
            pltpu.VMEM((CB, H), jnp.float32),
            pltpu.VMEM((CB, H), jnp.float32),
            pltpu.SemaphoreType.DMA,
            pltpu.SemaphoreType.DMA,
        ],
    )
    pool = l2(h, src2d, dst, wg)

    out = pl.pallas_call(
        _tc2_body,
        out_shape=jax.ShapeDtypeStruct((G, F_IN), jnp.float32),
    )(pool, stats, W2_l, b2.reshape(1, H), W2_r, Wlin,
      blin.reshape(1, F_IN))
    return out


# speedup vs baseline: 17.7530x; 1.2156x over previous
import jax
import jax.numpy as jnp
from jax import lax
from jax.experimental import pallas as pl
from jax.experimental.pallas import tpu as pltpu
from jax.experimental.pallas import tpu_sc as plsc

N = 50000
NPAD = 50048
E = 800000
F_IN = 26
FP = 32
H = 64
G = 64
NC = 2
NS = 16
NW = NC * NS
CB = 128
NCHUNKS = E // CB
CPT = NCHUNKS // NW
NLEFT = NCHUNKS - CPT * NW
IB = 13
NBLK = CPT // IB
ROWS_PER_TILE = NPAD // NS
ZROWS = 391
R = 3128
LANES = 16


def _l1_body(xpad, src2d, dst2d, out, acc, zbuf, srcl0, dstl0, srcl1, dstl1,
             stail, dtail, msgs0, msgs1, sem0, sem1, isem, zsem):
    c = lax.axis_index("c")
    s = lax.axis_index("s")
    w = c * NS + s

    def i_start(blk, sbuf, dbuf):
        cb0 = w * CPT + blk * IB
        pltpu.async_copy(src2d.at[pl.ds(cb0, IB), :], sbuf, isem)
        pltpu.async_copy(dst2d.at[pl.ds(cb0, IB), :], dbuf, isem)

    def i_wait(blk, sbuf, dbuf):
        cb0 = w * CPT + blk * IB
        pltpu.make_async_copy(src2d.at[pl.ds(cb0, IB), :], sbuf, isem).wait()
        pltpu.make_async_copy(dst2d.at[pl.ds(cb0, IB), :], dbuf, isem).wait()

    i_start(0, srcl0, dstl0)

    zero16 = jnp.zeros((LANES,), jnp.float32)
    for r in range(ZROWS):
        for j in range(FP // LANES):
            zbuf[r, pl.ds(j * LANES, LANES)] = zero16
    nz = ROWS_PER_TILE // ZROWS
    for i in range(nz):
        pltpu.async_copy(
            zbuf, acc.at[pl.ds(s * ROWS_PER_TILE + i * ZROWS, ZROWS), :],
            zsem)
    for i in range(nz):
        pltpu.make_async_copy(
            zbuf, acc.at[pl.ds(s * ROWS_PER_TILE + i * ZROWS, ZROWS), :],
            zsem).wait()
    plsc.subcore_barrier()

    def g_start(srcl, j, buf, sem):
        pltpu.async_copy(xpad.at[srcl.at[j]], buf, sem)

    def g_wait(srcl, j, buf, sem):
        pltpu.make_async_copy(xpad.at[srcl.at[j]], buf, sem).wait()

    def process(srcl, dstl):
        g_start(srcl, 0, msgs0, sem0)

        def body(t, carry2):
            c0 = 2 * t
            c1 = 2 * t + 1
            g_start(srcl, c1, msgs1, sem1)
            g_wait(srcl, c0, msgs0, sem0)
            pltpu.sync_copy(msgs0, acc.at[dstl.at[c0]], add=True)
            g_start(srcl, c0 + 2, msgs0, sem0)
            g_wait(srcl, c1, msgs1, sem1)
            pltpu.sync_copy(msgs1, acc.at[dstl.at[c1]], add=True)
            return carry2

        lax.fori_loop(0, IB // 2, body, 0)
        g_wait(srcl, IB - 1, msgs0, sem0)
        pltpu.sync_copy(msgs0, acc.at[dstl.at[IB - 1]], add=True)

    def blk_body(t, carry):
        b0 = 2 * t
        b1 = 2 * t + 1
        i_start(b1, srcl1, dstl1)
        i_wait(b0, srcl0, dstl0)
        process(srcl0, dstl0)

        @pl.when(b0 + 2 < NBLK)
        def _():
            i_start(b0 + 2, srcl0, dstl0)

        i_wait(b1, srcl1, dstl1)
        process(srcl1, dstl1)
        return carry

    lax.fori_loop(0, NBLK // 2, blk_body, 0)
    i_wait(NBLK - 1, srcl0, dstl0)
    process(srcl0, dstl0)

    @pl.when(w < NLEFT)
    def _():
        pltpu.sync_copy(src2d.at[NW * CPT + w], stail)
        pltpu.sync_copy(dst2d.at[NW * CPT + w], dtail)
        pltpu.async_copy(xpad.at[stail], msgs1, sem1).wait()
        pltpu.sync_copy(msgs1, acc.at[dtail], add=True)

    plsc.subcore_barrier()
    pltpu.sync_copy(
        acc.at[pl.ds(s * ROWS_PER_TILE, ROWS_PER_TILE), :],
        out.at[c, pl.ds(s * ROWS_PER_TILE, ROWS_PER_TILE), :])


def _scale_rows(msgs, dstall, wgtab, gbuf, base):
    for k in range(CB // LANES):
        dv = dstall[pl.ds(base + k * LANES, LANES)]
        pk = plsc.load_gather(wgtab, [dv])
        gv = pk & jnp.int32(63)
        wv = plsc.bitcast(pk & jnp.int32(-64), jnp.float32)
        gbuf[pl.ds(k * LANES, LANES)] = gv
        for e in range(LANES):
            ws = jnp.broadcast_to(wv[e], (LANES,))
            row = k * LANES + e
            for j in range(H // LANES):
                sl = pl.ds(j * LANES, LANES)
                msgs[row, sl] = msgs[row, sl] * ws


def _l2_body(h, src2d, dst1d, wg, out, acc, wgtab, srcl, dstall, zbuf,
             gbuf, stail, msgs0, msgs1, sem0, sem1):
    c = lax.axis_index("c")
    s = lax.axis_index("s")
    w = c * NS + s

    pltpu.sync_copy(src2d.at[pl.ds(w * CPT, CPT), :], srcl)
    pltpu.sync_copy(dst1d.at[pl.ds(w * CPT * CB, CPT * CB)], dstall)
    pltpu.sync_copy(wg, wgtab)

    def g_start(j, buf, sem):
        pltpu.async_copy(h.at[srcl.at[j]], buf, sem)

    def g_wait(j, buf, sem):
        pltpu.make_async_copy(h.at[srcl.at[j]], buf, sem).wait()

    g_start(0, msgs0, sem0)

    zero16 = jnp.zeros((LANES,), jnp.float32)
    for r in range(8):
        for j in range(H // LANES):
            zbuf[r, pl.ds(j * LANES, LANES)] = zero16

    @pl.when(s < G // 8)
    def _():
        pltpu.sync_copy(zbuf, acc.at[pl.ds(s * 8, 8), :])

    plsc.subcore_barrier()

    def consume(j, buf, sem):
        g_wait(j, buf, sem)
        _scale_rows(buf, dstall, wgtab, gbuf, j * CB)
        pltpu.sync_copy(buf, acc.at[gbuf], add=True)

    def body(t, carry):
        c0 = 2 * t
        c1 = 2 * t + 1
        g_start(c1, msgs1, sem1)
        consume(c0, msgs0, sem0)

        @pl.when(c0 + 2 < CPT)
        def _():
            g_start(c0 + 2, msgs0, sem0)

        consume(c1, msgs1, sem1)
        return carry

    lax.fori_loop(0, CPT // 2, body, 0)
    consume(CPT - 1, msgs0, sem0)

    @pl.when(w < NLEFT)
    def _():
        pltpu.sync_copy(src2d.at[NW * CPT + w], stail)
        pltpu.sync_copy(
            dst1d.at[pl.ds((NW * CPT + w) * CB, CB)],
            dstall.at[pl.ds(0, CB)])
        pltpu.async_copy(h.at[stail], msgs1, sem1).wait()
        _scale_rows(msgs1, dstall, wgtab, gbuf, 0)
        pltpu.sync_copy(msgs1, acc.at[gbuf], add=True)

    plsc.subcore_barrier()

    @pl.when(s == 0)
    def _():
        pltpu.sync_copy(acc, out.at[c])


def _tc1_body(p_ref, x_ref, batch_ref, wext_ref, h_ref, stats_ref, wg_ref):
    i = pl.program_id(0)
    agg = p_ref[0] + p_ref[1]
    deg = agg[:, 26:27]
    mean = agg / jnp.clip(deg, 1.0)
    inp = jnp.concatenate([mean, x_ref[...]], axis=1)
    hh = jnp.maximum(
        jnp.dot(inp, wext_ref[...], preferred_element_type=jnp.float32), 0.0)
    h_ref[...] = hh
    b = batch_ref[0, 0, :].reshape(R, 1)
    oh = (b == lax.broadcasted_iota(jnp.int32, (R, G), 1)).astype(jnp.float32)
    rhs = jnp.concatenate(
        [hh, jnp.ones((R, 1), jnp.float32),
         jnp.zeros((R, 128 - H - 1), jnp.float32)], axis=1)
    contrib = lax.dot_general(
        oh, rhs, (((0,), (0,)), ((), ())),
        preferred_element_type=jnp.float32)

    @pl.when(i == 0)
    def _():
        stats_ref[...] = jnp.zeros_like(stats_ref)

    stats_ref[...] += contrib

    wbits = lax.bitcast_convert_type(1.0 / jnp.clip(deg, 1.0), jnp.int32)
    packed = (wbits & jnp.int32(-64)) | b
    wg_ref[0, 0, :] = packed[:, 0]


def _tc2_body(pool_ref, stats_ref, w2l_ref, b2_ref, w2r_ref, wlin_ref,
              blin_ref, out_ref):
    agg2 = pool_ref[0] + pool_ref[1]
    segh = stats_ref[:, :H]
    counts = stats_ref[:, H:H + 1]
    ps = (jnp.dot(agg2, w2l_ref[...], preferred_element_type=jnp.float32)
          + jnp.dot(segh, w2r_ref[...], preferred_element_type=jnp.float32)
          + counts * b2_ref[...])
    pooled = ps / jnp.clip(counts, 1.0)
    out_ref[...] = (
        jnp.dot(pooled, wlin_ref[...], preferred_element_type=jnp.float32)
        + blin_ref[...])


def kernel(x, edge_index, batch, W1_l, b1, W1_r, W2_l, b2, W2_r, Wlin, blin):
    src = edge_index[0].astype(jnp.int32)
    dst = edge_index[1].astype(jnp.int32)
    batch = batch.astype(jnp.int32)
    src2d = src.reshape(NCHUNKS, CB)
    dst2d = dst.reshape(NCHUNKS, CB)

    xpad = jnp.pad(
        jnp.concatenate([x, jnp.ones((N, 1), jnp.float32)], axis=1),
        ((0, NPAD - N), (0, FP - F_IN - 1)))

    mesh = plsc.VectorSubcoreMesh(core_axis_name="c", subcore_axis_name="s")
    sc_params = pltpu.CompilerParams(use_tc_tiling_on_sc=False,
                                     needs_layout_passes=False)

    l1 = pl.kernel(
        _l1_body,
        out_type=jax.ShapeDtypeStruct((NC, NPAD, FP), jnp.float32),
        mesh=mesh,
        compiler_params=sc_params,
        scratch_types=[
            pltpu.VMEM_SHARED((NPAD, FP), jnp.float32),
            pltpu.VMEM((ZROWS, FP), jnp.float32),
            pltpu.VMEM((IB, CB), jnp.int32),
            pltpu.VMEM((IB, CB), jnp.int32),
            pltpu.VMEM((IB, CB), jnp.int32),
            pltpu.VMEM((IB, CB), jnp.int32),
            pltpu.VMEM((CB,), jnp.int32),
            pltpu.VMEM((CB,), jnp.int32),
            pltpu.VMEM((CB, FP), jnp.float32),
            pltpu.VMEM((CB, FP), jnp.float32),
            pltpu.SemaphoreType.DMA,
            pltpu.SemaphoreType.DMA,
            pltpu.SemaphoreType.DMA,
            pltpu.SemaphoreType.DMA,
        ],
    )
    partials = l1(xpad, src2d, dst2d)

    wext = jnp.zeros((2 * FP, H), jnp.float32)
    wext = wext.at[:F_IN, :].set(W1_l)
    wext = wext.at[FP:FP + F_IN, :].set(W1_r)
    wext = wext.at[FP + F_IN, :].set(b1)

    batch_pad = jnp.pad(batch, (0, NPAD - N), constant_values=G)
    batch3d = batch_pad.reshape(NPAD // R, 1, R)

    h, stats, wg3d = pl.pallas_call(
        _tc1_body,
        grid=(NPAD // R,),
        in_specs=[
            pl.BlockSpec((NC, R, FP), lambda i: (0, i, 0)),
            pl.BlockSpec((R, FP), lambda i: (i, 0)),
            pl.BlockSpec((1, 1, R), lambda i: (i, 0, 0)),
            pl.BlockSpec((2 * FP, H), lambda i: (0, 0)),
        ],
        out_specs=[
            pl.BlockSpec((R, H), lambda i: (i, 0)),
            pl.BlockSpec((G, 128), lambda i: (0, 0)),
            pl.BlockSpec((1, 1, R), lambda i: (i, 0, 0)),
        ],
        out_shape=[
            jax.ShapeDtypeStruct((NPAD, H), jnp.float32),
            jax.ShapeDtypeStruct((G, 128), jnp.float32),
            jax.ShapeDtypeStruct((NPAD // R, 1, R), jnp.int32),
        ],
    )(partials, xpad, batch3d, wext)
    wg = wg3d.reshape(NPAD)

    l2 = pl.kernel(
        _l2_body,
        out_type=jax.ShapeDtypeStruct((NC, G, H), jnp.float32),
        mesh=mesh,
        compiler_params=sc_params,
        scratch_types=[
            pltpu.VMEM_SHARED((G, H), jnp.float32),
            pltpu.VMEM((NPAD,), jnp.int32),
            pltpu.VMEM((CPT, CB), jnp.int32),
            pltpu.VMEM((CPT * CB,), jnp.int32),
            pltpu.VMEM((8, H), jnp.float32),
            pltpu.VMEM((CB,), jnp.int32),
            pltpu.VMEM((CB,), jnp.int32),
            pltpu.VMEM((CB, H), jnp.float32),
            pltpu.VMEM((CB, H), jnp.float32),
            pltpu.SemaphoreType.DMA,
            pltpu.SemaphoreType.DMA,
        ],
    )
    pool = l2(h, src2d, dst, wg)

    out = pl.pallas_call(
        _tc2_body,
        out_shape=jax.ShapeDtypeStruct((G, F_IN), jnp.float32),
    )(pool, stats, W2_l, b2.reshape(1, H), W2_r, Wlin,
      blin.reshape(1, F_IN))
    return out
